# Initial kernel scaffold; baseline (speedup 1.0000x reference)
import functools

import jax
import jax.numpy as jnp
from jax import lax
from jax.experimental import pallas as pl
from jax.experimental.pallas import tpu as pltpu
from jax.experimental.pallas import tpu_sc as plsc

_CB = 8192
_D = 32
_N = 16384
_BT = 256
_COMMIT = 0.25

_info = plsc.get_sparse_core_info()
_NC, _NS = _info.num_cores, _info.num_subcores
_NW = _NC * _NS
_BPW = _N // _NW


def _dist_argmin_body(x_ref, cbt_ref, idx_ref, loss_ref):
    i = pl.program_id(0)
    x = x_ref[...]
    cbt = cbt_ref[...]
    mm = lax.dot_general(x, cbt, (((1,), (0,)), ((), ())),
                         preferred_element_type=jnp.float32)
    x2 = jnp.sum(x * x, axis=1, keepdims=True)
    e2 = jnp.sum(cbt * cbt, axis=0, keepdims=True)
    dist = (x2 + e2) - 2.0 * mm
    idx_ref[...] = jnp.argmin(dist, axis=1).astype(jnp.int32)
    psum = jnp.sum(jnp.min(dist, axis=1))

    @pl.when(i == 0)
    def _init():
        loss_ref[0, 0] = 0.0

    loss_ref[0, 0] += psum


_sc_mesh = plsc.VectorSubcoreMesh(core_axis_name="c", subcore_axis_name="s")


@functools.partial(
    pl.kernel,
    out_type=jax.ShapeDtypeStruct((_N, _D), jnp.float32),
    mesh=_sc_mesh,
    scratch_types=[
        pltpu.VMEM((_BPW,), jnp.int32),
        pltpu.VMEM((_BPW, _D), jnp.float32),
        pltpu.SemaphoreType.DMA,
    ],
)
def _sc_gather(cb_hbm, idx_hbm, out_hbm, idx_v, rows_v, sem):
    wid = lax.axis_index("s") * _NC + lax.axis_index("c")
    base = wid * _BPW
    pltpu.sync_copy(idx_hbm.at[pl.ds(base, _BPW)], idx_v)
    pltpu.async_copy(cb_hbm.at[idx_v], rows_v, sem).wait()
    pltpu.sync_copy(rows_v, out_hbm.at[pl.ds(base, _BPW)])


def kernel(inputs, codebook):
    cbt = codebook.T
    idx, loss_sum = pl.pallas_call(
        _dist_argmin_body,
        grid=(_N // _BT,),
        in_specs=[
            pl.BlockSpec((_BT, _D), lambda i: (i, 0)),
            pl.BlockSpec((_D, _CB), lambda i: (0, 0)),
        ],
        out_specs=[
            pl.BlockSpec((_BT,), lambda i: (i,)),
            pl.BlockSpec((1, 1), lambda i: (0, 0)),
        ],
        out_shape=[
            jax.ShapeDtypeStruct((_N,), jnp.int32),
            jax.ShapeDtypeStruct((1, 1), jnp.float32),
        ],
    )(inputs, cbt)
    quantized = _sc_gather(codebook, idx)
    loss = loss_sum[0, 0] * (_COMMIT / (_N * _D))
    return quantized, loss, idx


# speedup vs baseline: 1.2600x; 1.2600x over previous
import functools

import jax
import jax.numpy as jnp
from jax import lax
from jax.experimental import pallas as pl
from jax.experimental.pallas import tpu as pltpu
from jax.experimental.pallas import tpu_sc as plsc

_CB = 8192
_D = 32
_N = 16384
_BT = 256
_COMMIT = 0.25

_NC, _NS = 2, 16
_NW = _NC * _NS
_BPW = _N // _NW


_ARG_TILE = 4096


def _rtne_bf16(v):
    i = lax.bitcast_convert_type(v, jnp.int32)
    odd = lax.shift_right_logical(i, 16) & jnp.int32(1)
    r = (i + jnp.int32(0x7FFF) + odd) & jnp.int32(-65536)
    return lax.bitcast_convert_type(r, jnp.float32)


def _dist_argmin_body(x_ref, cbt_ref, idx_ref, loss_ref):
    i = pl.program_id(0)
    x = x_ref[...]
    cbt = cbt_ref[...]
    mm = lax.dot_general(x, cbt, (((1,), (0,)), ((), ())),
                         preferred_element_type=jnp.float32)
    x2 = jnp.sum(x * x, axis=1, keepdims=True)
    e2 = jnp.sum(cbt * cbt, axis=0, keepdims=True)
    dist = (x2 + e2) - 2.0 * mm

    acc_v = jnp.full((x.shape[0],), jnp.inf, jnp.float32)
    acc_i = jnp.zeros((x.shape[0],), jnp.int32)
    for n in range(_CB // _ARG_TILE):
        tile = dist[:, n * _ARG_TILE:(n + 1) * _ARG_TILE]
        tv = jnp.min(tile, axis=1)
        iota = lax.broadcasted_iota(jnp.int32, tile.shape, 1)
        ti = jnp.min(jnp.where(tile == tv[:, None], iota, jnp.int32(2**30)),
                     axis=1) + jnp.int32(n * _ARG_TILE)
        better = tv < acc_v
        acc_v = _rtne_bf16(jnp.where(better, tv, acc_v))
        acc_i = jnp.where(better, ti, acc_i)
    idx_ref[...] = acc_i

    psum = jnp.sum(jnp.min(dist, axis=1)).reshape(1, 1)

    @pl.when(i == 0)
    def _init():
        loss_ref[...] = jnp.zeros((1, 1), jnp.float32)

    loss_ref[...] += psum


@functools.cache
def _make_sc_gather():
    mesh = plsc.VectorSubcoreMesh(core_axis_name="c", subcore_axis_name="s")

    @functools.partial(
        pl.kernel,
        out_type=jax.ShapeDtypeStruct((_N, _D), jnp.float32),
        mesh=mesh,
        compiler_params=pltpu.CompilerParams(use_tc_tiling_on_sc=False),
        scratch_types=[
            pltpu.VMEM((_BPW,), jnp.int32),
            pltpu.VMEM((_BPW, _D), jnp.float32),
            pltpu.SemaphoreType.DMA,
        ],
    )
    def _sc_gather(cb_hbm, idx_hbm, out_hbm, idx_v, rows_v, sem):
        wid = lax.axis_index("s") * _NC + lax.axis_index("c")
        base = wid * _BPW
        pltpu.sync_copy(idx_hbm.at[pl.ds(base, _BPW)], idx_v)
        pltpu.async_copy(cb_hbm.at[idx_v], rows_v, sem).wait()
        pltpu.sync_copy(rows_v, out_hbm.at[pl.ds(base, _BPW)])

    return _sc_gather


def kernel(inputs, codebook):
    cbt = codebook.T
    idx, loss_sum = pl.pallas_call(
        _dist_argmin_body,
        grid=(_N // _BT,),
        in_specs=[
            pl.BlockSpec((_BT, _D), lambda i: (i, 0)),
            pl.BlockSpec((_D, _CB), lambda i: (0, 0)),
        ],
        out_specs=[
            pl.BlockSpec((_BT,), lambda i: (i,)),
            pl.BlockSpec((1, 1), lambda i: (0, 0)),
        ],
        out_shape=[
            jax.ShapeDtypeStruct((_N,), jnp.int32),
            jax.ShapeDtypeStruct((1, 1), jnp.float32),
        ],
    )(inputs, cbt)
    quantized = _make_sc_gather()(codebook, idx)
    loss = loss_sum[0, 0] * (_COMMIT / (_N * _D))
    return quantized, loss, idx


# speedup vs baseline: 1.3519x; 1.0730x over previous
import functools

import jax
import jax.numpy as jnp
from jax import lax
from jax.experimental import pallas as pl
from jax.experimental.pallas import tpu as pltpu
from jax.experimental.pallas import tpu_sc as plsc

_CB = 8192
_D = 32
_N = 16384
_BT = 256
_COMMIT = 0.25

_NC, _NS = 2, 16
_NW = _NC * _NS
_BPW = _N // _NW


_ARG_TILE = 4096


def _rtne_bf16(v):
    i = lax.bitcast_convert_type(v, jnp.int32)
    odd = lax.shift_right_logical(i, 16) & jnp.int32(1)
    r = (i + jnp.int32(0x7FFF) + odd) & jnp.int32(-65536)
    return lax.bitcast_convert_type(r, jnp.float32)


def _dist_argmin_body(x_ref, cbt_ref, idx_ref, loss_ref):
    i = pl.program_id(0)
    x = x_ref[...]
    cbt = cbt_ref[...]
    mm = lax.dot_general(x, cbt, (((1,), (0,)), ((), ())),
                         preferred_element_type=jnp.float32)
    x2 = jnp.sum(x * x, axis=1, keepdims=True)
    e2 = jnp.sum(cbt * cbt, axis=0, keepdims=True)
    dist = (x2 + e2) - 2.0 * mm

    acc_v = jnp.full((x.shape[0],), jnp.inf, jnp.float32)
    acc_i = jnp.zeros((x.shape[0],), jnp.int32)
    true_min = jnp.full((x.shape[0],), jnp.inf, jnp.float32)
    for n in range(_CB // _ARG_TILE):
        tile = dist[:, n * _ARG_TILE:(n + 1) * _ARG_TILE]
        tv = jnp.min(tile, axis=1)
        iota = lax.broadcasted_iota(jnp.int32, tile.shape, 1)
        ti = jnp.min(jnp.where(tile == tv[:, None], iota, jnp.int32(2**30)),
                     axis=1) + jnp.int32(n * _ARG_TILE)
        better = tv < acc_v
        acc_v = _rtne_bf16(jnp.where(better, tv, acc_v))
        acc_i = jnp.where(better, ti, acc_i)
        true_min = jnp.minimum(true_min, tv)
    idx_ref[...] = acc_i

    psum = jnp.sum(true_min).reshape(1, 1)

    @pl.when(i == 0)
    def _init():
        loss_ref[...] = jnp.zeros((1, 1), jnp.float32)

    loss_ref[...] += psum


@functools.cache
def _make_sc_gather():
    mesh = plsc.VectorSubcoreMesh(core_axis_name="c", subcore_axis_name="s")

    @functools.partial(
        pl.kernel,
        out_type=jax.ShapeDtypeStruct((_N, _D), jnp.float32),
        mesh=mesh,
        compiler_params=pltpu.CompilerParams(use_tc_tiling_on_sc=False),
        scratch_types=[
            pltpu.VMEM((_BPW,), jnp.int32),
            pltpu.VMEM((_BPW, _D), jnp.float32),
            pltpu.SemaphoreType.DMA,
        ],
    )
    def _sc_gather(cb_hbm, idx_hbm, out_hbm, idx_v, rows_v, sem):
        wid = lax.axis_index("s") * _NC + lax.axis_index("c")
        base = wid * _BPW
        pltpu.sync_copy(idx_hbm.at[pl.ds(base, _BPW)], idx_v)
        pltpu.async_copy(cb_hbm.at[idx_v], rows_v, sem).wait()
        pltpu.sync_copy(rows_v, out_hbm.at[pl.ds(base, _BPW)])

    return _sc_gather


def kernel(inputs, codebook):
    cbt = codebook.T
    idx, loss_sum = pl.pallas_call(
        _dist_argmin_body,
        grid=(_N // _BT,),
        in_specs=[
            pl.BlockSpec((_BT, _D), lambda i: (i, 0)),
            pl.BlockSpec((_D, _CB), lambda i: (0, 0)),
        ],
        out_specs=[
            pl.BlockSpec((_BT,), lambda i: (i,)),
            pl.BlockSpec((1, 1), lambda i: (0, 0)),
        ],
        out_shape=[
            jax.ShapeDtypeStruct((_N,), jnp.int32),
            jax.ShapeDtypeStruct((1, 1), jnp.float32),
        ],
    )(inputs, cbt)
    quantized = _make_sc_gather()(codebook, idx)
    loss = loss_sum[0, 0] * (_COMMIT / (_N * _D))
    return quantized, loss, idx


# speedup vs baseline: 1.3882x; 1.0268x over previous
import functools

import jax
import jax.numpy as jnp
from jax import lax
from jax.experimental import pallas as pl
from jax.experimental.pallas import tpu as pltpu
from jax.experimental.pallas import tpu_sc as plsc

_CB = 8192
_D = 32
_N = 16384
_BT = 512
_COMMIT = 0.25

_NC, _NS = 2, 16
_NW = _NC * _NS
_BPW = _N // _NW


_ARG_TILE = 4096


def _rtne_bf16(v):
    i = lax.bitcast_convert_type(v, jnp.int32)
    odd = lax.shift_right_logical(i, 16) & jnp.int32(1)
    r = (i + jnp.int32(0x7FFF) + odd) & jnp.int32(-65536)
    return lax.bitcast_convert_type(r, jnp.float32)


def _dist_argmin_body(x_ref, cbt_ref, idx_ref, loss_ref):
    i = pl.program_id(0)
    x = x_ref[...]
    cbt = cbt_ref[...]
    mm = lax.dot_general(x, cbt, (((1,), (0,)), ((), ())),
                         preferred_element_type=jnp.float32)
    x2 = jnp.sum(x * x, axis=1, keepdims=True)
    e2 = jnp.sum(cbt * cbt, axis=0, keepdims=True)
    dist = (x2 + e2) - 2.0 * mm

    acc_v = jnp.full((x.shape[0],), jnp.inf, jnp.float32)
    acc_i = jnp.zeros((x.shape[0],), jnp.int32)
    true_min = jnp.full((x.shape[0],), jnp.inf, jnp.float32)
    for n in range(_CB // _ARG_TILE):
        tile = dist[:, n * _ARG_TILE:(n + 1) * _ARG_TILE]
        tv = jnp.min(tile, axis=1)
        iota = lax.broadcasted_iota(jnp.int32, tile.shape, 1)
        ti = jnp.min(jnp.where(tile == tv[:, None], iota, jnp.int32(2**30)),
                     axis=1) + jnp.int32(n * _ARG_TILE)
        better = tv < acc_v
        acc_v = _rtne_bf16(jnp.where(better, tv, acc_v))
        acc_i = jnp.where(better, ti, acc_i)
        true_min = jnp.minimum(true_min, tv)
    idx_ref[...] = acc_i

    psum = jnp.sum(true_min).reshape(1, 1)

    @pl.when(i == 0)
    def _init():
        loss_ref[...] = jnp.zeros((1, 1), jnp.float32)

    loss_ref[...] += psum


@functools.cache
def _make_sc_gather():
    mesh = plsc.VectorSubcoreMesh(core_axis_name="c", subcore_axis_name="s")

    @functools.partial(
        pl.kernel,
        out_type=jax.ShapeDtypeStruct((_N, _D), jnp.float32),
        mesh=mesh,
        compiler_params=pltpu.CompilerParams(use_tc_tiling_on_sc=False),
        scratch_types=[
            pltpu.VMEM((_BPW,), jnp.int32),
            pltpu.VMEM((_BPW, _D), jnp.float32),
            pltpu.SemaphoreType.DMA,
        ],
    )
    def _sc_gather(cb_hbm, idx_hbm, out_hbm, idx_v, rows_v, sem):
        wid = lax.axis_index("s") * _NC + lax.axis_index("c")
        base = wid * _BPW
        pltpu.sync_copy(idx_hbm.at[pl.ds(base, _BPW)], idx_v)
        pltpu.async_copy(cb_hbm.at[idx_v], rows_v, sem).wait()
        pltpu.sync_copy(rows_v, out_hbm.at[pl.ds(base, _BPW)])

    return _sc_gather


def kernel(inputs, codebook):
    cbt = codebook.T
    idx, loss_sum = pl.pallas_call(
        _dist_argmin_body,
        grid=(_N // _BT,),
        in_specs=[
            pl.BlockSpec((_BT, _D), lambda i: (i, 0)),
            pl.BlockSpec((_D, _CB), lambda i: (0, 0)),
        ],
        out_specs=[
            pl.BlockSpec((_BT,), lambda i: (i,)),
            pl.BlockSpec((1, 1), lambda i: (0, 0)),
        ],
        out_shape=[
            jax.ShapeDtypeStruct((_N,), jnp.int32),
            jax.ShapeDtypeStruct((1, 1), jnp.float32),
        ],
    )(inputs, cbt)
    quantized = _make_sc_gather()(codebook, idx)
    loss = loss_sum[0, 0] * (_COMMIT / (_N * _D))
    return quantized, loss, idx


# speedup vs baseline: 1.4527x; 1.0465x over previous
import functools

import jax
import jax.numpy as jnp
from jax import lax
from jax.experimental import pallas as pl
from jax.experimental.pallas import tpu as pltpu
from jax.experimental.pallas import tpu_sc as plsc

_CB = 8192
_D = 32
_N = 16384
_BT = 512
_COMMIT = 0.25

_NC, _NS = 2, 16
_NW = _NC * _NS
_BPW = _N // _NW


_ARG_TILE = 4096


def _rtne_bf16(v):
    i = lax.bitcast_convert_type(v, jnp.int32)
    odd = lax.shift_right_logical(i, 16) & jnp.int32(1)
    r = (i + jnp.int32(0x7FFF) + odd) & jnp.int32(-65536)
    return lax.bitcast_convert_type(r, jnp.float32)


def _dist_argmin_body(x_ref, cbt_ref, idx_ref, loss_ref):
    i = pl.program_id(0)
    x = x_ref[...]
    cbt = cbt_ref[...]
    mm = lax.dot_general(x, cbt, (((1,), (0,)), ((), ())),
                         preferred_element_type=jnp.float32)
    x2 = jnp.sum(x * x, axis=1, keepdims=True)
    e2 = jnp.sum(cbt * cbt, axis=0, keepdims=True)
    dist = (x2 + e2) - 2.0 * mm

    acc_v = jnp.full((x.shape[0],), jnp.inf, jnp.float32)
    acc_i = jnp.zeros((x.shape[0],), jnp.int32)
    true_min = jnp.full((x.shape[0],), jnp.inf, jnp.float32)
    for n in range(_CB // _ARG_TILE):
        tile = dist[:, n * _ARG_TILE:(n + 1) * _ARG_TILE]
        tv = jnp.min(tile, axis=1)
        ti = jnp.argmin(tile, axis=1).astype(jnp.int32) + jnp.int32(n * _ARG_TILE)
        better = tv < acc_v
        acc_v = _rtne_bf16(jnp.where(better, tv, acc_v))
        acc_i = jnp.where(better, ti, acc_i)
        true_min = jnp.minimum(true_min, tv)
    idx_ref[...] = acc_i

    psum = jnp.sum(true_min).reshape(1, 1)

    @pl.when(i == 0)
    def _init():
        loss_ref[...] = jnp.zeros((1, 1), jnp.float32)

    loss_ref[...] += psum


@functools.cache
def _make_sc_gather():
    mesh = plsc.VectorSubcoreMesh(core_axis_name="c", subcore_axis_name="s")

    @functools.partial(
        pl.kernel,
        out_type=jax.ShapeDtypeStruct((_N, _D), jnp.float32),
        mesh=mesh,
        compiler_params=pltpu.CompilerParams(use_tc_tiling_on_sc=False),
        scratch_types=[
            pltpu.VMEM((_BPW,), jnp.int32),
            pltpu.VMEM((_BPW, _D), jnp.float32),
            pltpu.SemaphoreType.DMA,
        ],
    )
    def _sc_gather(cb_hbm, idx_hbm, out_hbm, idx_v, rows_v, sem):
        wid = lax.axis_index("s") * _NC + lax.axis_index("c")
        base = wid * _BPW
        pltpu.sync_copy(idx_hbm.at[pl.ds(base, _BPW)], idx_v)
        pltpu.async_copy(cb_hbm.at[idx_v], rows_v, sem).wait()
        pltpu.sync_copy(rows_v, out_hbm.at[pl.ds(base, _BPW)])

    return _sc_gather


def kernel(inputs, codebook):
    cbt = codebook.T
    idx, loss_sum = pl.pallas_call(
        _dist_argmin_body,
        grid=(_N // _BT,),
        in_specs=[
            pl.BlockSpec((_BT, _D), lambda i: (i, 0)),
            pl.BlockSpec((_D, _CB), lambda i: (0, 0)),
        ],
        out_specs=[
            pl.BlockSpec((_BT,), lambda i: (i,)),
            pl.BlockSpec((1, 1), lambda i: (0, 0)),
        ],
        out_shape=[
            jax.ShapeDtypeStruct((_N,), jnp.int32),
            jax.ShapeDtypeStruct((1, 1), jnp.float32),
        ],
    )(inputs, cbt)
    quantized = _make_sc_gather()(codebook, idx)
    loss = loss_sum[0, 0] * (_COMMIT / (_N * _D))
    return quantized, loss, idx
